# baseline (speedup 1.0000x reference)
import functools
import math

import jax
import jax.numpy as jnp
from jax import lax
from jax.experimental import pallas as pl
from jax.experimental.pallas import tpu as pltpu
from jax.experimental.pallas import tpu_sc as plsc

D_MODEL = 64
SCALE = math.sqrt(D_MODEL)
NUM_WORKERS = 32
SUB = 128
PADDED_D = 128
LANES = 16


def _transpose_scale(rbuf, tbuf, row_iotas):
    iota = row_iotas[0]

    @plsc.parallel_loop(0, D_MODEL, unroll=4)
    def body(c):
        col = (c + iota) & (D_MODEL - 1)
        for k in range(SUB // LANES):
            v = plsc.load_gather(rbuf, [row_iotas[k], col])
            plsc.store_scatter(tbuf, [col, row_iotas[k]], v * SCALE)


def _make_sc_gather(batch):
    n_per_w = batch // NUM_WORKERS
    n_sub = n_per_w // SUB
    b_total = 4096

    mesh = plsc.VectorSubcoreMesh(core_axis_name="c", subcore_axis_name="s")

    @functools.partial(
        pl.kernel,
        out_type=jax.ShapeDtypeStruct((batch // b_total * D_MODEL, b_total),
                                      jnp.float32),
        mesh=mesh,
        compiler_params=pltpu.CompilerParams(use_tc_tiling_on_sc=True,
                                             needs_layout_passes=False),
        scratch_types=[
            pltpu.VMEM((n_sub, SUB), jnp.int32),
            pltpu.VMEM((SUB, PADDED_D), jnp.float32),
            pltpu.VMEM((SUB, PADDED_D), jnp.float32),
            pltpu.VMEM((D_MODEL, SUB), jnp.float32),
            pltpu.VMEM((D_MODEL, SUB), jnp.float32),
            pltpu.SemaphoreType.DMA,
            pltpu.SemaphoreType.DMA,
            pltpu.SemaphoreType.DMA,
            pltpu.SemaphoreType.DMA,
            pltpu.SemaphoreType.DMA,
        ],
    )
    def gather_kernel(idx_hbm, table_hbm, out_hbm, idx_v, rows0, rows1,
                      tr0, tr1, isem, gsem0, gsem1, osem0, osem1):
        wid = lax.axis_index("s") * 2 + lax.axis_index("c")

        pltpu.async_copy(idx_hbm.at[wid], idx_v, isem).wait()

        rbufs = (rows0, rows1)
        tbufs = (tr0, tr1)
        gsems = (gsem0, gsem1)
        osems = (osem0, osem1)
        row_iotas = [lax.iota(jnp.int32, LANES) + k * LANES
                     for k in range(SUB // LANES)]

        def out_slice(g):
            mg = wid * n_sub + g
            return out_hbm.at[pl.ds((mg // 32) * D_MODEL, D_MODEL),
                              pl.ds((mg % 32) * SUB, SUB)]

        def issue_gather(g, rbuf, par):
            pltpu.async_copy(table_hbm.at[idx_v.at[g]], rbuf, gsems[par])

        def wait_gather(g, rbuf, par):
            pltpu.make_async_copy(
                table_hbm.at[idx_v.at[g]], rbuf, gsems[par]).wait()

        def issue_out(g, tbuf, par):
            pltpu.async_copy(tbuf, out_slice(g), osems[par])

        def wait_out(g, tbuf, par):
            pltpu.make_async_copy(tbuf, out_slice(g), osems[par]).wait()

        def step(g, b, issue_next):
            if issue_next:
                issue_gather(g + 1, rbufs[1 - b], 1 - b)
            wait_gather(g, rbufs[b], b)
            wait_out(g - 2, tbufs[b], b)
            _transpose_scale(rbufs[b], tbufs[b], row_iotas)
            issue_out(g, tbufs[b], b)

        def step_nowaitout(g, b, issue_next):
            if issue_next:
                issue_gather(g + 1, rbufs[1 - b], 1 - b)
            wait_gather(g, rbufs[b], b)
            _transpose_scale(rbufs[b], tbufs[b], row_iotas)
            issue_out(g, tbufs[b], b)

        issue_gather(0, rbufs[0], 0)
        step_nowaitout(0, 0, True)
        step_nowaitout(1, 1, True)

        def body(g2, _):
            step(2 * g2, 0, True)
            step(2 * g2 + 1, 1, True)
            return 0

        lax.fori_loop(1, n_sub // 2 - 1, body, 0)

        step(n_sub - 2, 0, True)
        step(n_sub - 1, 1, False)
        wait_out(n_sub - 2, tbufs[0], 0)
        wait_out(n_sub - 1, tbufs[1], 1)

    return gather_kernel


def kernel(x, table):
    batch = x.size
    idx = x.T.reshape(NUM_WORKERS, batch // NUM_WORKERS // SUB, SUB)
    idx = idx.astype(jnp.int32)
    table_p = jnp.pad(table, ((0, 0), (0, PADDED_D - D_MODEL)))
    out_t = _make_sc_gather(batch)(idx, table_p)
    return out_t.reshape(x.shape[1], D_MODEL, x.shape[0]).transpose(2, 0, 1)


# speedup vs baseline: 1.5631x; 1.5631x over previous
import functools
import math

import jax
import jax.numpy as jnp
from jax import lax
from jax.experimental import pallas as pl
from jax.experimental.pallas import tpu as pltpu
from jax.experimental.pallas import tpu_sc as plsc

D_MODEL = 64
SCALE = math.sqrt(D_MODEL)
NUM_WORKERS = 32
SUB = 128
PADDED_D = 128
LANES = 16


def _transpose_block(rbuf, tbuf, row_iotas):
    iota = row_iotas[0]

    @plsc.parallel_loop(0, D_MODEL, unroll=4)
    def body(c):
        col = (c + iota) & (D_MODEL - 1)
        for k in range(SUB // LANES):
            v = plsc.load_gather(rbuf, [row_iotas[k], col])
            plsc.store_scatter(tbuf, [col, row_iotas[k]], v)


def _make_tc_prep(n_rows):
    blk = 16384

    def prep(in_ref, out_ref):
        out_ref[:, pl.ds(0, D_MODEL)] = in_ref[...].T * SCALE

    return pl.pallas_call(
        prep,
        grid=(pl.cdiv(n_rows, blk),),
        in_specs=[pl.BlockSpec((D_MODEL, blk), lambda i: (0, i))],
        out_specs=pl.BlockSpec((blk, PADDED_D), lambda i: (i, 0)),
        out_shape=jax.ShapeDtypeStruct((n_rows, PADDED_D), jnp.float32),
    )


def _make_sc_gather(batch):
    n_per_w = batch // NUM_WORKERS
    n_sub = n_per_w // SUB
    b_total = 4096

    mesh = plsc.VectorSubcoreMesh(core_axis_name="c", subcore_axis_name="s")

    @functools.partial(
        pl.kernel,
        out_type=jax.ShapeDtypeStruct((batch // b_total * D_MODEL, b_total),
                                      jnp.float32),
        mesh=mesh,
        compiler_params=pltpu.CompilerParams(use_tc_tiling_on_sc=True,
                                             needs_layout_passes=False),
        scratch_types=[
            pltpu.VMEM((n_sub, SUB), jnp.int32),
            pltpu.VMEM((SUB, PADDED_D), jnp.float32),
            pltpu.VMEM((SUB, PADDED_D), jnp.float32),
            pltpu.VMEM((D_MODEL, SUB), jnp.float32),
            pltpu.VMEM((D_MODEL, SUB), jnp.float32),
            pltpu.SemaphoreType.DMA,
            pltpu.SemaphoreType.DMA,
            pltpu.SemaphoreType.DMA,
            pltpu.SemaphoreType.DMA,
            pltpu.SemaphoreType.DMA,
        ],
    )
    def gather_kernel(idx_hbm, table_hbm, out_hbm, idx_v, rows0, rows1,
                      tr0, tr1, isem, gsem0, gsem1, osem0, osem1):
        wid = lax.axis_index("s") * 2 + lax.axis_index("c")

        pltpu.async_copy(idx_hbm.at[wid], idx_v, isem).wait()

        rbufs = (rows0, rows1)
        tbufs = (tr0, tr1)
        gsems = (gsem0, gsem1)
        osems = (osem0, osem1)
        row_iotas = [lax.iota(jnp.int32, LANES) + k * LANES
                     for k in range(SUB // LANES)]

        def out_slice(g):
            mg = wid * n_sub + g
            return out_hbm.at[pl.ds((mg // 32) * D_MODEL, D_MODEL),
                              pl.ds((mg % 32) * SUB, SUB)]

        def issue_gather(g, rbuf, par):
            pltpu.async_copy(table_hbm.at[idx_v.at[g]], rbuf, gsems[par])

        def wait_gather(g, rbuf, par):
            pltpu.make_async_copy(
                table_hbm.at[idx_v.at[g]], rbuf, gsems[par]).wait()

        def issue_out(g, tbuf, par):
            pltpu.async_copy(tbuf, out_slice(g), osems[par])

        def wait_out(g, tbuf, par):
            pltpu.make_async_copy(tbuf, out_slice(g), osems[par]).wait()

        def step(g, b, issue_next):
            if issue_next:
                issue_gather(g + 1, rbufs[1 - b], 1 - b)
            wait_gather(g, rbufs[b], b)
            wait_out(g - 2, tbufs[b], b)
            _transpose_block(rbufs[b], tbufs[b], row_iotas)
            issue_out(g, tbufs[b], b)

        def step_nowaitout(g, b, issue_next):
            if issue_next:
                issue_gather(g + 1, rbufs[1 - b], 1 - b)
            wait_gather(g, rbufs[b], b)
            _transpose_block(rbufs[b], tbufs[b], row_iotas)
            issue_out(g, tbufs[b], b)

        issue_gather(0, rbufs[0], 0)
        step_nowaitout(0, 0, True)
        step_nowaitout(1, 1, True)

        def body(g2, _):
            step(2 * g2, 0, True)
            step(2 * g2 + 1, 1, True)
            return 0

        lax.fori_loop(1, n_sub // 2 - 1, body, 0)

        step(n_sub - 2, 0, True)
        step(n_sub - 1, 1, False)
        wait_out(n_sub - 2, tbufs[0], 0)
        wait_out(n_sub - 1, tbufs[1], 1)

    return gather_kernel


def kernel(x, table):
    batch = x.size
    idx = x.T.reshape(NUM_WORKERS, batch // NUM_WORKERS // SUB, SUB)
    idx = idx.astype(jnp.int32)
    table_p = _make_tc_prep(table.shape[0])(table.T)
    out_t = _make_sc_gather(batch)(idx, table_p)
    return out_t.reshape(x.shape[1], D_MODEL, x.shape[0]).transpose(2, 0, 1)


# speedup vs baseline: 1.5865x; 1.0150x over previous
import functools
import math

import jax
import jax.numpy as jnp
from jax import lax
from jax.experimental import pallas as pl
from jax.experimental.pallas import tpu as pltpu
from jax.experimental.pallas import tpu_sc as plsc

D_MODEL = 64
SCALE = math.sqrt(D_MODEL)
NUM_WORKERS = 32
SUB = 128
PADDED_D = 128
LANES = 16


def _transpose_block(rbuf, tbuf, row_iotas):
    iota = row_iotas[0]

    @plsc.parallel_loop(0, D_MODEL, unroll=4)
    def body(c):
        col = (c + iota) & (D_MODEL - 1)
        for k in range(SUB // LANES):
            v = plsc.load_gather(rbuf, [row_iotas[k], col])
            plsc.store_scatter(tbuf, [col, row_iotas[k]], v)


def _make_tc_prep(n_rows):
    blk = 32768

    def prep(in_ref, out_ref):
        out_ref[:, pl.ds(0, D_MODEL)] = in_ref[...].T * SCALE

    return pl.pallas_call(
        prep,
        grid=(pl.cdiv(n_rows, blk),),
        in_specs=[pl.BlockSpec((D_MODEL, blk), lambda i: (0, i))],
        out_specs=pl.BlockSpec((blk, PADDED_D), lambda i: (i, 0)),
        out_shape=jax.ShapeDtypeStruct((n_rows, PADDED_D), jnp.float32),
    )


def _make_sc_gather(batch):
    n_per_w = batch // NUM_WORKERS
    n_sub = n_per_w // SUB
    b_total = 4096

    mesh = plsc.VectorSubcoreMesh(core_axis_name="c", subcore_axis_name="s")

    @functools.partial(
        pl.kernel,
        out_type=jax.ShapeDtypeStruct((batch // b_total * D_MODEL, b_total),
                                      jnp.float32),
        mesh=mesh,
        compiler_params=pltpu.CompilerParams(use_tc_tiling_on_sc=True,
                                             needs_layout_passes=False),
        scratch_types=[
            pltpu.VMEM((n_sub, SUB), jnp.int32),
            pltpu.VMEM((SUB, PADDED_D), jnp.float32),
            pltpu.VMEM((SUB, PADDED_D), jnp.float32),
            pltpu.VMEM((D_MODEL, SUB), jnp.float32),
            pltpu.VMEM((D_MODEL, SUB), jnp.float32),
            pltpu.SemaphoreType.DMA,
            pltpu.SemaphoreType.DMA,
            pltpu.SemaphoreType.DMA,
            pltpu.SemaphoreType.DMA,
            pltpu.SemaphoreType.DMA,
        ],
    )
    def gather_kernel(idx_hbm, table_hbm, out_hbm, idx_v, rows0, rows1,
                      tr0, tr1, isem, gsem0, gsem1, osem0, osem1):
        wid = lax.axis_index("s") * 2 + lax.axis_index("c")

        pltpu.async_copy(idx_hbm.at[wid], idx_v, isem).wait()

        rbufs = (rows0, rows1)
        tbufs = (tr0, tr1)
        gsems = (gsem0, gsem1)
        osems = (osem0, osem1)
        row_iotas = [lax.iota(jnp.int32, LANES) + k * LANES
                     for k in range(SUB // LANES)]

        def out_slice(g):
            mg = wid * n_sub + g
            return out_hbm.at[pl.ds((mg // 32) * D_MODEL, D_MODEL),
                              pl.ds((mg % 32) * SUB, SUB)]

        def issue_gather(g, rbuf, par):
            pltpu.async_copy(table_hbm.at[idx_v.at[g]], rbuf, gsems[par])

        def wait_gather(g, rbuf, par):
            pltpu.make_async_copy(
                table_hbm.at[idx_v.at[g]], rbuf, gsems[par]).wait()

        def issue_out(g, tbuf, par):
            pltpu.async_copy(tbuf, out_slice(g), osems[par])

        def wait_out(g, tbuf, par):
            pltpu.make_async_copy(tbuf, out_slice(g), osems[par]).wait()

        def step(g, b, issue_next):
            if issue_next:
                issue_gather(g + 1, rbufs[1 - b], 1 - b)
            wait_gather(g, rbufs[b], b)
            wait_out(g - 2, tbufs[b], b)
            _transpose_block(rbufs[b], tbufs[b], row_iotas)
            issue_out(g, tbufs[b], b)

        def step_nowaitout(g, b, issue_next):
            if issue_next:
                issue_gather(g + 1, rbufs[1 - b], 1 - b)
            wait_gather(g, rbufs[b], b)
            _transpose_block(rbufs[b], tbufs[b], row_iotas)
            issue_out(g, tbufs[b], b)

        issue_gather(0, rbufs[0], 0)
        step_nowaitout(0, 0, True)
        step_nowaitout(1, 1, True)

        def body(g2, _):
            step(2 * g2, 0, True)
            step(2 * g2 + 1, 1, True)
            return 0

        lax.fori_loop(1, n_sub // 2 - 1, body, 0)

        step(n_sub - 2, 0, True)
        step(n_sub - 1, 1, False)
        wait_out(n_sub - 2, tbufs[0], 0)
        wait_out(n_sub - 1, tbufs[1], 1)

    return gather_kernel


def kernel(x, table):
    batch = x.size
    idx = x.T.reshape(NUM_WORKERS, batch // NUM_WORKERS // SUB, SUB)
    idx = idx.astype(jnp.int32)
    table_p = _make_tc_prep(table.shape[0])(table.T)
    out_t = _make_sc_gather(batch)(idx, table_p)
    return out_t.reshape(x.shape[1], D_MODEL, x.shape[0]).transpose(2, 0, 1)
